# baseline (speedup 1.0000x reference)
import functools

import jax
import jax.numpy as jnp
from jax import lax
from jax.experimental import pallas as pl
from jax.experimental.pallas import tpu as pltpu
from jax.experimental.pallas import tpu_sc as plsc

B, S = 4096, 50
E = 128
NW = 32
TOTAL = B * S
PER_W = TOTAL // NW
CHUNK = 128
NJ = PER_W // CHUNK
NBUF = 5
NSTEPS = NJ // NBUF


def _make_sc_gather():
    mesh = plsc.VectorSubcoreMesh(core_axis_name="c", subcore_axis_name="s")

    @functools.partial(
        pl.kernel,
        mesh=mesh,
        out_type=jax.ShapeDtypeStruct((TOTAL, E), jnp.float32),
        scratch_types=(
            [pltpu.VMEM((NJ, CHUNK), jnp.int32)]
            + [pltpu.VMEM((CHUNK, E), jnp.float32) for _ in range(NBUF)]
            + [pltpu.SemaphoreType.DMA for _ in range(2 * NBUF)]
        ),
    )
    def k(idx_hbm, table_hbm, out_hbm, idx_v, *rest):
        bufs = rest[:NBUF]
        gsem = rest[NBUF:2 * NBUF]
        osem = rest[2 * NBUF:]
        wid = lax.axis_index("s") * 2 + lax.axis_index("c")
        base = wid * PER_W
        pltpu.sync_copy(idx_hbm.at[wid], idx_v)

        def dst(j):
            return out_hbm.at[pl.ds(base + j * CHUNK, CHUNK)]

        for b in range(NBUF):
            pltpu.async_copy(table_hbm.at[idx_v.at[b]], bufs[b], gsem[b])

        def body(i, carry):
            j0 = i * NBUF
            for b in range(NBUF):
                j = j0 + b
                pltpu.make_async_copy(
                    table_hbm.at[idx_v.at[j]], bufs[b], gsem[b]).wait()
                pltpu.async_copy(bufs[b], dst(j), osem[b])
            for b in range(NBUF):
                j = j0 + b
                pltpu.make_async_copy(bufs[b], dst(j), osem[b]).wait()
                pltpu.async_copy(
                    table_hbm.at[idx_v.at[j + NBUF]], bufs[b], gsem[b])
            return carry

        lax.fori_loop(0, NSTEPS - 1, body, 0)

        j0 = (NSTEPS - 1) * NBUF
        for b in range(NBUF):
            j = j0 + b
            pltpu.make_async_copy(
                table_hbm.at[idx_v.at[j]], bufs[b], gsem[b]).wait()
            pltpu.async_copy(bufs[b], dst(j), osem[b])
        for b in range(NBUF):
            j = j0 + b
            pltpu.make_async_copy(bufs[b], dst(j), osem[b]).wait()

    return k


RB = 16


def _relayout_body(i_ref, o_ref):
    o_ref[...] = i_ref[...].reshape(RB, S, E)


_tc_relayout = pl.pallas_call(
    _relayout_body,
    grid=(B // RB,),
    in_specs=[pl.BlockSpec((RB * S, E), lambda i: (i, 0))],
    out_specs=pl.BlockSpec((RB, S, E), lambda i: (i, 0, 0)),
    out_shape=jax.ShapeDtypeStruct((B, S, E), jnp.float32),
)

_sc_gather = _make_sc_gather()


@jax.jit
def kernel(x, table):
    idx = x.reshape(NW, NJ, CHUNK)
    inter = _sc_gather(idx, table)
    return _tc_relayout(inter)
